# baseline (speedup 1.0000x reference)
import functools

import jax
import jax.numpy as jnp
from jax import lax
from jax.experimental import pallas as pl
from jax.experimental.pallas import tpu as pltpu
from jax.experimental.pallas import tpu_sc as plsc

_D = 128
_N_NODES = 131072
_N_EDGES = 524288
_N_OUT = 1 + _N_NODES + _N_EDGES
_CHUNK = 128
_NC, _NS = 2, 16
_NW = _NC * _NS
_E_PER_W = _N_EDGES // _NW
_EC_PER_W = _E_PER_W // _CHUNK
_N_PER_W = _N_NODES // _NW
_NCH_PER_W = _N_PER_W // _CHUNK
_NT = _D // 16
_NB = 4

_R_WE = 0
_R_BE = 4
_R_WN = 8
_R_BN = 12


def _body(node_f, edge_f, tok2, et2, ord2, table, consts, gt, out,
          cv, iv, ev, ov, fv, bv, sx, gtv, sg, sf, so):
    w = lax.axis_index("s") * _NC + lax.axis_index("c")

    pltpu.sync_copy(consts, cv)
    pltpu.sync_copy(tok2.at[pl.ds(w * _EC_PER_W, _EC_PER_W)], iv)
    pltpu.sync_copy(et2.at[pl.ds(w * _EC_PER_W, _EC_PER_W)], ev)
    pltpu.sync_copy(ord2.at[pl.ds(w * _NCH_PER_W, _NCH_PER_W)], ov)

    iot = lax.iota(jnp.int32, 16)

    def drain_write(b):
        pltpu.make_async_copy(
            out.at[pl.ds(0, _CHUNK)], bv.at[b], so.at[b]).wait()

    def fire_write(b, row_base):
        for q in range(_NT):
            sx[b, pl.ds(q * 16, 16)] = row_base + (iot + q * 16)
        pltpu.async_copy(bv.at[b], out.at[sx.at[b]], so.at[b])

    we = [[cv[_R_WE + k, pl.ds(t * 16, 16)] for t in range(_NT)]
          for k in range(4)]

    def fire_edge_inputs(j, b):
        c = w * _EC_PER_W + j
        pltpu.async_copy(edge_f.at[c], fv.at[b], sf.at[b])
        pltpu.async_copy(table.at[iv.at[j]], bv.at[b], sg.at[b])

    fire_edge_inputs(0, 0)

    def edge_chunk(j, carry):
        b = jnp.bitwise_and(j, _NB - 1)
        nb = jnp.bitwise_and(j + 1, _NB - 1)
        c = w * _EC_PER_W + j

        @pl.when(j + 1 < _EC_PER_W)
        def _():
            @pl.when(j >= _NB - 1)
            def _():
                drain_write(nb)
            fire_edge_inputs(j + 1, nb)

        pltpu.make_async_copy(edge_f.at[0], fv.at[b], sf.at[b]).wait()
        pltpu.make_async_copy(
            table.at[pl.ds(0, _CHUNK)], bv.at[b], sg.at[b]).wait()

        @plsc.parallel_loop(0, _CHUNK // 16, unroll=4)
        def group(g):
            slg = pl.ds(g * 16, 16)
            fvec = [fv[b, k, slg] for k in range(4)]
            evec = ev[j, slg]
            for i in range(16):
                f0, f1, f2, f3 = (fvec[k][i] for k in range(4))
                e = evec[i]
                r = g * 16 + i
                for t in range(_NT):
                    sl = pl.ds(t * 16, 16)
                    s0 = we[0][t] * f0 + we[1][t] * f1
                    s1 = we[2][t] * f2 + we[3][t] * f3
                    val = (s0 + s1) + cv[_R_BE + e, sl]
                    plsc.addupdate(bv.at[b, r, sl], val)

        fire_write(b, 1 + _N_NODES + c * _CHUNK)
        return carry

    lax.fori_loop(0, _EC_PER_W, edge_chunk, 0)
    for b in range(_NB):
        drain_write(b)

    wn = [[cv[_R_WN + k, pl.ds(t * 16, 16)] for t in range(_NT)]
          for k in range(4)]

    def fire_node_inputs(j, b):
        c = w * _NCH_PER_W + j
        pltpu.async_copy(node_f.at[c], fv.at[b], sf.at[b])

    fire_node_inputs(0, 0)

    def node_chunk(j, carry):
        b = jnp.bitwise_and(j, _NB - 1)
        nb = jnp.bitwise_and(j + 1, _NB - 1)
        c = w * _NCH_PER_W + j

        @pl.when(j + 1 < _NCH_PER_W)
        def _():
            @pl.when(j >= _NB - 1)
            def _():
                drain_write(nb)
            fire_node_inputs(j + 1, nb)

        pltpu.make_async_copy(node_f.at[0], fv.at[b], sf.at[b]).wait()

        @plsc.parallel_loop(0, _CHUNK // 16, unroll=2)
        def group(g):
            slg = pl.ds(g * 16, 16)
            fvec = [fv[b, k, slg] for k in range(4)]
            ovec = ov[j, slg]
            for i in range(16):
                f0, f1, f2, f3 = (fvec[k][i] for k in range(4))
                o = ovec[i]
                r = g * 16 + i
                for t in range(_NT):
                    sl = pl.ds(t * 16, 16)
                    s0 = wn[0][t] * f0 + wn[1][t] * f1
                    s1 = wn[2][t] * f2 + wn[3][t] * f3
                    bv[b, r, sl] = (s0 + s1) + cv[_R_BN + o, sl]

        fire_write(b, 1 + c * _CHUNK)
        return carry

    lax.fori_loop(0, _NCH_PER_W, node_chunk, 0)
    for b in range(_NB):
        drain_write(b)

    @pl.when(w == 0)
    def _():
        pltpu.sync_copy(gt, gtv)
        pltpu.sync_copy(gtv, out.at[pl.ds(0, 1)])


@jax.jit
def _fused_sc(node_f, edge_f, tok2, et2, ord2, table, consts, gt):
    kern = functools.partial(
        pl.kernel,
        mesh=plsc.VectorSubcoreMesh(core_axis_name="c", subcore_axis_name="s"),
        out_type=jax.ShapeDtypeStruct((_N_OUT, _D), jnp.float32),
        scratch_types=[
            pltpu.VMEM((16, _D), jnp.float32),
            pltpu.VMEM((_EC_PER_W, _CHUNK), jnp.int32),
            pltpu.VMEM((_EC_PER_W, _CHUNK), jnp.int32),
            pltpu.VMEM((_NCH_PER_W, _CHUNK), jnp.int32),
            pltpu.VMEM((_NB, 4, _CHUNK), jnp.float32),
            pltpu.VMEM((_NB, _CHUNK, _D), jnp.float32),
            pltpu.VMEM((_NB, _CHUNK), jnp.int32),
            pltpu.VMEM((1, _D), jnp.float32),
            pltpu.SemaphoreType.DMA((_NB,)),
            pltpu.SemaphoreType.DMA((_NB,)),
            pltpu.SemaphoreType.DMA((_NB,)),
        ],
    )(_body)
    return kern(node_f, edge_f, tok2, et2, ord2, table, consts, gt)


def kernel(node_features, edge_features, token_ids, etype_ids, order_ids,
           W_node, b_node, W_edge, b_edge,
           token_table, etype_table, order_table, graph_token):
    consts = jnp.concatenate([
        W_edge.T,
        b_edge[None, :] + etype_table,
        W_node.T,
        b_node[None, :] + order_table,
        jnp.zeros((1, _D), jnp.float32),
    ], axis=0)
    tok2 = token_ids.reshape(_N_EDGES // _CHUNK, _CHUNK)
    et2 = etype_ids.reshape(_N_EDGES // _CHUNK, _CHUNK)
    ord2 = order_ids.reshape(_N_NODES // _CHUNK, _CHUNK)
    nf_r = node_features.T.reshape(4, _N_NODES // _CHUNK, _CHUNK).transpose(1, 0, 2)
    ef_r = edge_features.T.reshape(4, _N_EDGES // _CHUNK, _CHUNK).transpose(1, 0, 2)
    return _fused_sc(nf_r, ef_r, tok2, et2, ord2, token_table, consts,
                     graph_token.reshape(1, _D))


# speedup vs baseline: 1.3102x; 1.3102x over previous
import functools

import jax
import jax.numpy as jnp
from jax import lax
from jax.experimental import pallas as pl
from jax.experimental.pallas import tpu as pltpu
from jax.experimental.pallas import tpu_sc as plsc

_D = 128
_N_NODES = 131072
_N_EDGES = 524288
_N_OUT = 1 + _N_NODES + _N_EDGES
_CHUNK = 128
_NC, _NS = 2, 16
_NW = _NC * _NS
_E_PER_W = _N_EDGES // _NW
_EC_PER_W = _E_PER_W // _CHUNK
_N_PER_W = _N_NODES // _NW
_NCH_PER_W = _N_PER_W // _CHUNK
_NT = _D // 16
_NB = 4

_R_WE = 0
_R_BE = 4
_R_WN = 8
_R_BN = 12


def _body(node_f, edge_f, tok2, et2, ord2, table, consts, gt, out,
          cv, iv, ev, ov, fv, bv, sx, gtv, sg, sf, so):
    w = lax.axis_index("s") * _NC + lax.axis_index("c")

    pltpu.sync_copy(consts, cv)
    pltpu.sync_copy(tok2.at[pl.ds(w * _EC_PER_W, _EC_PER_W)], iv)
    pltpu.sync_copy(et2.at[pl.ds(w * _EC_PER_W, _EC_PER_W)], ev)
    pltpu.sync_copy(ord2.at[pl.ds(w * _NCH_PER_W, _NCH_PER_W)], ov)

    iot = lax.iota(jnp.int32, 16)

    def drain_write(b):
        pltpu.make_async_copy(
            out.at[pl.ds(0, _CHUNK)], bv.at[b], so.at[b]).wait()

    def fire_write(b, row_base):
        for q in range(_NT):
            sx[b, pl.ds(q * 16, 16)] = row_base + (iot + q * 16)
        pltpu.async_copy(bv.at[b], out.at[sx.at[b]], so.at[b])

    we = [[cv[_R_WE + k, pl.ds(t * 16, 16)] for t in range(_NT)]
          for k in range(4)]

    def fire_edge_inputs(j, b):
        c = w * _EC_PER_W + j
        pltpu.async_copy(edge_f.at[c], fv.at[b], sf.at[b])
        pltpu.async_copy(table.at[iv.at[j]], bv.at[b], sg.at[b])

    fire_edge_inputs(0, 0)

    def edge_chunk(j, carry):
        b = jnp.bitwise_and(j, _NB - 1)
        nb = jnp.bitwise_and(j + 1, _NB - 1)
        c = w * _EC_PER_W + j

        @pl.when(j + 1 < _EC_PER_W)
        def _():
            @pl.when(j >= _NB - 1)
            def _():
                drain_write(nb)
            fire_edge_inputs(j + 1, nb)

        pltpu.make_async_copy(edge_f.at[0], fv.at[b], sf.at[b]).wait()
        pltpu.make_async_copy(
            table.at[pl.ds(0, _CHUNK)], bv.at[b], sg.at[b]).wait()

        @plsc.parallel_loop(0, _CHUNK // 16, unroll=1)
        def group(g):
            slg = pl.ds(g * 16, 16)
            fvec = [fv[b, k, slg] for k in range(4)]
            evec = ev[j, slg]
            for i in range(16):
                f0, f1, f2, f3 = (fvec[k][i] for k in range(4))
                e = evec[i]
                r = g * 16 + i
                for t in range(_NT):
                    sl = pl.ds(t * 16, 16)
                    s0 = we[0][t] * f0 + we[1][t] * f1
                    s1 = we[2][t] * f2 + we[3][t] * f3
                    val = (s0 + s1) + cv[_R_BE + e, sl]
                    plsc.addupdate(bv.at[b, r, sl], val)

        fire_write(b, 1 + _N_NODES + c * _CHUNK)
        return carry

    lax.fori_loop(0, _EC_PER_W, edge_chunk, 0)
    for b in range(_NB):
        drain_write(b)

    wn = [[cv[_R_WN + k, pl.ds(t * 16, 16)] for t in range(_NT)]
          for k in range(4)]

    def fire_node_inputs(j, b):
        c = w * _NCH_PER_W + j
        pltpu.async_copy(node_f.at[c], fv.at[b], sf.at[b])

    fire_node_inputs(0, 0)

    def node_chunk(j, carry):
        b = jnp.bitwise_and(j, _NB - 1)
        nb = jnp.bitwise_and(j + 1, _NB - 1)
        c = w * _NCH_PER_W + j

        @pl.when(j + 1 < _NCH_PER_W)
        def _():
            @pl.when(j >= _NB - 1)
            def _():
                drain_write(nb)
            fire_node_inputs(j + 1, nb)

        pltpu.make_async_copy(node_f.at[0], fv.at[b], sf.at[b]).wait()

        @plsc.parallel_loop(0, _CHUNK // 16, unroll=1)
        def group(g):
            slg = pl.ds(g * 16, 16)
            fvec = [fv[b, k, slg] for k in range(4)]
            ovec = ov[j, slg]
            for i in range(16):
                f0, f1, f2, f3 = (fvec[k][i] for k in range(4))
                o = ovec[i]
                r = g * 16 + i
                for t in range(_NT):
                    sl = pl.ds(t * 16, 16)
                    s0 = wn[0][t] * f0 + wn[1][t] * f1
                    s1 = wn[2][t] * f2 + wn[3][t] * f3
                    bv[b, r, sl] = (s0 + s1) + cv[_R_BN + o, sl]

        fire_write(b, 1 + c * _CHUNK)
        return carry

    lax.fori_loop(0, _NCH_PER_W, node_chunk, 0)
    for b in range(_NB):
        drain_write(b)

    @pl.when(w == 0)
    def _():
        pltpu.sync_copy(gt, gtv)
        pltpu.sync_copy(gtv, out.at[pl.ds(0, 1)])


@jax.jit
def _fused_sc(node_f, edge_f, tok2, et2, ord2, table, consts, gt):
    kern = functools.partial(
        pl.kernel,
        mesh=plsc.VectorSubcoreMesh(core_axis_name="c", subcore_axis_name="s"),
        out_type=jax.ShapeDtypeStruct((_N_OUT, _D), jnp.float32),
        scratch_types=[
            pltpu.VMEM((16, _D), jnp.float32),
            pltpu.VMEM((_EC_PER_W, _CHUNK), jnp.int32),
            pltpu.VMEM((_EC_PER_W, _CHUNK), jnp.int32),
            pltpu.VMEM((_NCH_PER_W, _CHUNK), jnp.int32),
            pltpu.VMEM((_NB, 4, _CHUNK), jnp.float32),
            pltpu.VMEM((_NB, _CHUNK, _D), jnp.float32),
            pltpu.VMEM((_NB, _CHUNK), jnp.int32),
            pltpu.VMEM((1, _D), jnp.float32),
            pltpu.SemaphoreType.DMA((_NB,)),
            pltpu.SemaphoreType.DMA((_NB,)),
            pltpu.SemaphoreType.DMA((_NB,)),
        ],
    )(_body)
    return kern(node_f, edge_f, tok2, et2, ord2, table, consts, gt)


def kernel(node_features, edge_features, token_ids, etype_ids, order_ids,
           W_node, b_node, W_edge, b_edge,
           token_table, etype_table, order_table, graph_token):
    consts = jnp.concatenate([
        W_edge.T,
        b_edge[None, :] + etype_table,
        W_node.T,
        b_node[None, :] + order_table,
        jnp.zeros((1, _D), jnp.float32),
    ], axis=0)
    tok2 = token_ids.reshape(_N_EDGES // _CHUNK, _CHUNK)
    et2 = etype_ids.reshape(_N_EDGES // _CHUNK, _CHUNK)
    ord2 = order_ids.reshape(_N_NODES // _CHUNK, _CHUNK)
    nf_r = node_features.T.reshape(4, _N_NODES // _CHUNK, _CHUNK).transpose(1, 0, 2)
    ef_r = edge_features.T.reshape(4, _N_EDGES // _CHUNK, _CHUNK).transpose(1, 0, 2)
    return _fused_sc(nf_r, ef_r, tok2, et2, ord2, token_table, consts,
                     graph_token.reshape(1, _D))


# speedup vs baseline: 1.9615x; 1.4971x over previous
import functools

import jax
import jax.numpy as jnp
from jax import lax
from jax.experimental import pallas as pl
from jax.experimental.pallas import tpu as pltpu
from jax.experimental.pallas import tpu_sc as plsc

_D = 128
_N_NODES = 131072
_N_EDGES = 524288
_N_OUT = 1 + _N_NODES + _N_EDGES
_CHUNK = 128
_NC, _NS = 2, 16
_NW = _NC * _NS
_E_PER_W = _N_EDGES // _NW
_EC_PER_W = _E_PER_W // _CHUNK
_N_PER_W = _N_NODES // _NW
_NCH_PER_W = _N_PER_W // _CHUNK
_NT = _D // 16
_NB = 4

_R_WE = 0
_R_BE = 4
_R_WN = 8
_R_BN = 12


def _body(node_f, edge_f, tok2, et2, ord2, table, consts, gt, out,
          cv, iv, ev, ov, fv, bv, sx, gtv, sg, sf, so):
    w = lax.axis_index("s") * _NC + lax.axis_index("c")

    pltpu.sync_copy(consts, cv)
    pltpu.sync_copy(tok2.at[pl.ds(w * _EC_PER_W, _EC_PER_W)], iv)
    pltpu.sync_copy(et2.at[pl.ds(w * _EC_PER_W, _EC_PER_W)], ev)
    pltpu.sync_copy(ord2.at[pl.ds(w * _NCH_PER_W, _NCH_PER_W)], ov)

    iot = lax.iota(jnp.int32, 16)

    def drain_write(b):
        pltpu.make_async_copy(
            out.at[pl.ds(0, _CHUNK)], bv.at[b], so.at[b]).wait()

    def fire_write(b, row_base):
        for q in range(_NT):
            sx[b, pl.ds(q * 16, 16)] = row_base + (iot + q * 16)
        pltpu.async_copy(bv.at[b], out.at[sx.at[b]], so.at[b])

    we = [[cv[_R_WE + k, pl.ds(t * 16, 16)] for t in range(_NT)]
          for k in range(4)]

    def fire_edge_inputs(j, b):
        c = w * _EC_PER_W + j
        pltpu.async_copy(edge_f.at[c], fv.at[b], sf.at[b])
        pltpu.async_copy(table.at[iv.at[j]], bv.at[b], sg.at[b])

    fire_edge_inputs(0, 0)
    fire_edge_inputs(1, 1)

    def edge_chunk(j, carry):
        b = jnp.bitwise_and(j, _NB - 1)
        nb = jnp.bitwise_and(j + 2, _NB - 1)
        c = w * _EC_PER_W + j

        @pl.when(j + 2 < _EC_PER_W)
        def _():
            @pl.when(j >= 2)
            def _():
                drain_write(nb)
            fire_edge_inputs(j + 2, nb)

        pltpu.make_async_copy(edge_f.at[0], fv.at[b], sf.at[b]).wait()
        pltpu.make_async_copy(
            table.at[pl.ds(0, _CHUNK)], bv.at[b], sg.at[b]).wait()

        @plsc.parallel_loop(0, _CHUNK // 16, unroll=2)
        def group(g):
            slg = pl.ds(g * 16, 16)
            fvec = [fv[b, k, slg] for k in range(4)]
            evec = ev[j, slg]
            for i in range(16):
                f0, f1, f2, f3 = (fvec[k][i] for k in range(4))
                e = evec[i]
                r = g * 16 + i
                for t in range(_NT):
                    sl = pl.ds(t * 16, 16)
                    s0 = we[0][t] * f0 + we[1][t] * f1
                    s1 = we[2][t] * f2 + we[3][t] * f3
                    val = (s0 + s1) + cv[_R_BE + e, sl]
                    plsc.addupdate(bv.at[b, r, sl], val)

        fire_write(b, 1 + _N_NODES + c * _CHUNK)
        return carry

    lax.fori_loop(0, _EC_PER_W, edge_chunk, 0)
    for b in range(_NB):
        drain_write(b)

    wn = [[cv[_R_WN + k, pl.ds(t * 16, 16)] for t in range(_NT)]
          for k in range(4)]

    def fire_node_inputs(j, b):
        c = w * _NCH_PER_W + j
        pltpu.async_copy(node_f.at[c], fv.at[b], sf.at[b])

    fire_node_inputs(0, 0)
    fire_node_inputs(1, 1)

    def node_chunk(j, carry):
        b = jnp.bitwise_and(j, _NB - 1)
        nb = jnp.bitwise_and(j + 2, _NB - 1)
        c = w * _NCH_PER_W + j

        @pl.when(j + 2 < _NCH_PER_W)
        def _():
            @pl.when(j >= 2)
            def _():
                drain_write(nb)
            fire_node_inputs(j + 2, nb)

        pltpu.make_async_copy(node_f.at[0], fv.at[b], sf.at[b]).wait()

        @plsc.parallel_loop(0, _CHUNK // 16, unroll=2)
        def group(g):
            slg = pl.ds(g * 16, 16)
            fvec = [fv[b, k, slg] for k in range(4)]
            ovec = ov[j, slg]
            for i in range(16):
                f0, f1, f2, f3 = (fvec[k][i] for k in range(4))
                o = ovec[i]
                r = g * 16 + i
                for t in range(_NT):
                    sl = pl.ds(t * 16, 16)
                    s0 = wn[0][t] * f0 + wn[1][t] * f1
                    s1 = wn[2][t] * f2 + wn[3][t] * f3
                    bv[b, r, sl] = (s0 + s1) + cv[_R_BN + o, sl]

        fire_write(b, 1 + c * _CHUNK)
        return carry

    lax.fori_loop(0, _NCH_PER_W, node_chunk, 0)
    for b in range(_NB):
        drain_write(b)

    @pl.when(w == 0)
    def _():
        pltpu.sync_copy(gt, gtv)
        pltpu.sync_copy(gtv, out.at[pl.ds(0, 1)])


@jax.jit
def _fused_sc(node_f, edge_f, tok2, et2, ord2, table, consts, gt):
    kern = functools.partial(
        pl.kernel,
        mesh=plsc.VectorSubcoreMesh(core_axis_name="c", subcore_axis_name="s"),
        out_type=jax.ShapeDtypeStruct((_N_OUT, _D), jnp.float32),
        scratch_types=[
            pltpu.VMEM((16, _D), jnp.float32),
            pltpu.VMEM((_EC_PER_W, _CHUNK), jnp.int32),
            pltpu.VMEM((_EC_PER_W, _CHUNK), jnp.int32),
            pltpu.VMEM((_NCH_PER_W, _CHUNK), jnp.int32),
            pltpu.VMEM((_NB, 4, _CHUNK), jnp.float32),
            pltpu.VMEM((_NB, _CHUNK, _D), jnp.float32),
            pltpu.VMEM((_NB, _CHUNK), jnp.int32),
            pltpu.VMEM((1, _D), jnp.float32),
            pltpu.SemaphoreType.DMA((_NB,)),
            pltpu.SemaphoreType.DMA((_NB,)),
            pltpu.SemaphoreType.DMA((_NB,)),
        ],
    )(_body)
    return kern(node_f, edge_f, tok2, et2, ord2, table, consts, gt)


def kernel(node_features, edge_features, token_ids, etype_ids, order_ids,
           W_node, b_node, W_edge, b_edge,
           token_table, etype_table, order_table, graph_token):
    consts = jnp.concatenate([
        W_edge.T,
        b_edge[None, :] + etype_table,
        W_node.T,
        b_node[None, :] + order_table,
        jnp.zeros((1, _D), jnp.float32),
    ], axis=0)
    tok2 = token_ids.reshape(_N_EDGES // _CHUNK, _CHUNK)
    et2 = etype_ids.reshape(_N_EDGES // _CHUNK, _CHUNK)
    ord2 = order_ids.reshape(_N_NODES // _CHUNK, _CHUNK)
    nf_r = node_features.T.reshape(4, _N_NODES // _CHUNK, _CHUNK).transpose(1, 0, 2)
    ef_r = edge_features.T.reshape(4, _N_EDGES // _CHUNK, _CHUNK).transpose(1, 0, 2)
    return _fused_sc(nf_r, ef_r, tok2, et2, ord2, token_table, consts,
                     graph_token.reshape(1, _D))
